# Initial kernel scaffold; baseline (speedup 1.0000x reference)
import jax
import jax.numpy as jnp
from jax.experimental import pallas as pl


def kernel(x, edge_type, mul_w, bias_w, w_edge_w, w1, b1, w2, b2, ln_w, ln_b):
    raise NotImplementedError("write your pallas kernel here")


# speedup vs baseline: 5.5924x; 5.5924x over previous
import functools

import jax
import jax.numpy as jnp
from jax import lax
from jax.experimental import pallas as pl
from jax.experimental.pallas import tpu as pltpu
from jax.experimental.pallas import tpu_sc as plsc

_K = 128
_HID = 256
_EPS = 1e-5
_NC = 2
_NS = 16
_NW = _NC * _NS
_CH = 512


def _sc_gather(table, aux_table, idx2):
    p_total = idx2.size
    nch = idx2.shape[0] // _NW
    bpw = p_total // _NW
    mesh = plsc.VectorSubcoreMesh(core_axis_name="c", subcore_axis_name="s")

    @functools.partial(
        pl.kernel,
        out_type=[
            jax.ShapeDtypeStruct((p_total, _K), table.dtype),
            jax.ShapeDtypeStruct((p_total, 16), aux_table.dtype),
        ],
        mesh=mesh,
        scratch_types=[
            pltpu.VMEM((nch, _CH), jnp.int32),
            pltpu.VMEM((_CH, _K), table.dtype),
            pltpu.VMEM((_CH, 16), aux_table.dtype),
            pltpu.SemaphoreType.DMA,
            pltpu.SemaphoreType.DMA,
        ],
        compiler_params=pltpu.CompilerParams(use_tc_tiling_on_sc=False),
    )
    def k(table_hbm, aux_hbm, idx_hbm, g_hbm, a_hbm, idx_v, rows_v, aux_v,
          sem1, sem2):
        wid = lax.axis_index("s") * _NC + lax.axis_index("c")
        base = wid * bpw
        pltpu.sync_copy(idx_hbm.at[pl.ds(wid * nch, nch)], idx_v)

        @pl.loop(0, nch)
        def _(j):
            c1 = pltpu.async_copy(table_hbm.at[idx_v.at[j]], rows_v, sem1)
            c2 = pltpu.async_copy(aux_hbm.at[idx_v.at[j]], aux_v, sem2)
            c1.wait()
            c2.wait()
            pltpu.sync_copy(rows_v, g_hbm.at[pl.ds(base + j * _CH, _CH)])
            pltpu.sync_copy(aux_v, a_hbm.at[pl.ds(base + j * _CH, _CH)])

    return k(table, aux_table, idx2)


def _gelu_exact(z):
    a = z * 0.7071067811865476
    s = jnp.sign(a)
    u = jnp.abs(a)
    t = 1.0 / (1.0 + 0.3275911 * u)
    p = t * (0.254829592
             + t * (-0.284496736
                    + t * (1.421413741
                           + t * (-1.453152027 + t * 1.061405429))))
    erf = s * (1.0 - p * jnp.exp(-u * u))
    return 0.5 * z * (1.0 + erf)


def _tc_body(x_ref, g_ref, aux_ref, w1_ref, b1_ref, w2_ref, b2_ref,
             lnw_ref, lnb_ref, o_ref):
    xb = x_ref[...]
    h = xb * w1_ref[...] + b1_ref[...]
    h = _gelu_exact(h)
    d = jnp.dot(h, w2_ref[...], preferred_element_type=jnp.float32)
    d = d + b2_ref[...]
    mu = jnp.mean(d, axis=-1, keepdims=True)
    c = d - mu
    var = jnp.mean(c * c, axis=-1, keepdims=True)
    dn = c * lax.rsqrt(var + _EPS) * lnw_ref[...] + lnb_ref[...]
    gate = 1.0 / (1.0 + jnp.exp(-(aux_ref[:, 0:1] * xb + aux_ref[:, 1:2])))
    o_ref[...] = dn + g_ref[...].astype(jnp.float32) * gate


def _tc_fused(x2, g, aux, w1, b1, w2, b2, ln_w, ln_b, r_block, interpret=False):
    p_total = x2.shape[0]
    grid = (p_total // r_block,)
    return pl.pallas_call(
        _tc_body,
        grid=grid,
        in_specs=[
            pl.BlockSpec((r_block, 1), lambda i: (i, 0)),
            pl.BlockSpec((r_block, _K), lambda i: (i, 0)),
            pl.BlockSpec((r_block, 16), lambda i: (i, 0)),
            pl.BlockSpec((1, _HID), lambda i: (0, 0)),
            pl.BlockSpec((1, _HID), lambda i: (0, 0)),
            pl.BlockSpec((_HID, _K), lambda i: (0, 0)),
            pl.BlockSpec((1, _K), lambda i: (0, 0)),
            pl.BlockSpec((1, _K), lambda i: (0, 0)),
            pl.BlockSpec((1, _K), lambda i: (0, 0)),
        ],
        out_specs=pl.BlockSpec((r_block, _K), lambda i: (i, 0)),
        out_shape=jax.ShapeDtypeStruct((p_total, _K), jnp.float32),
        interpret=interpret,
    )(x2, g, aux, w1, b1.reshape(1, _HID), w2, b2.reshape(1, _K),
      ln_w.reshape(1, _K), ln_b.reshape(1, _K))


def kernel(x, edge_type, mul_w, bias_w, w_edge_w, w1, b1, w2, b2, ln_w, ln_b):
    b, n, _ = x.shape
    p_total = b * n * n
    vocab = w_edge_w.shape[0]
    idx = edge_type.reshape(-1).astype(jnp.int32)
    idx2 = idx.reshape(_NW * (p_total // (_NW * _CH)), _CH)
    aux_table = jnp.concatenate(
        [mul_w, bias_w, jnp.zeros((vocab, 14), jnp.float32)], axis=1)
    g, aux = _sc_gather(w_edge_w, aux_table, idx2)
    x2 = x.reshape(p_total, 1)
    out = _tc_fused(x2, g, aux, w1, b1, w2, b2, ln_w, ln_b, r_block=512)
    return out.reshape(b, n, n, _K)


# speedup vs baseline: 6.3283x; 1.1316x over previous
import functools

import jax
import jax.numpy as jnp
from jax import lax
from jax.experimental import pallas as pl
from jax.experimental.pallas import tpu as pltpu
from jax.experimental.pallas import tpu_sc as plsc

_K = 128
_HID = 256
_EPS = 1e-5
_NC = 2
_NS = 16
_NW = _NC * _NS
_CH = 512


def _sc_gather(table, aux_table, idx2):
    p_total = idx2.size
    nch = idx2.shape[0] // _NW
    bpw = p_total // _NW
    mesh = plsc.VectorSubcoreMesh(core_axis_name="c", subcore_axis_name="s")

    @functools.partial(
        pl.kernel,
        out_type=[
            jax.ShapeDtypeStruct((p_total, _K), table.dtype),
            jax.ShapeDtypeStruct((p_total, 16), aux_table.dtype),
        ],
        mesh=mesh,
        scratch_types=[
            pltpu.VMEM((nch, _CH), jnp.int32),
            pltpu.VMEM((_CH, _K), table.dtype),
            pltpu.VMEM((_CH, 16), aux_table.dtype),
            pltpu.SemaphoreType.DMA,
            pltpu.SemaphoreType.DMA,
        ],
        compiler_params=pltpu.CompilerParams(use_tc_tiling_on_sc=False),
    )
    def k(table_hbm, aux_hbm, idx_hbm, g_hbm, a_hbm, idx_v, rows_v, aux_v,
          sem1, sem2):
        wid = lax.axis_index("s") * _NC + lax.axis_index("c")
        base = wid * bpw
        pltpu.sync_copy(idx_hbm.at[pl.ds(wid * nch, nch)], idx_v)

        @pl.loop(0, nch)
        def _(j):
            c1 = pltpu.async_copy(table_hbm.at[idx_v.at[j]], rows_v, sem1)
            c2 = pltpu.async_copy(aux_hbm.at[idx_v.at[j]], aux_v, sem2)
            c1.wait()
            c2.wait()
            pltpu.sync_copy(rows_v, g_hbm.at[pl.ds(base + j * _CH, _CH)])
            pltpu.sync_copy(aux_v, a_hbm.at[pl.ds(base + j * _CH, _CH)])

    return k(table, aux_table, idx2)


def _gelu(z):
    dt = z.dtype
    z2 = z * z
    u = z * (dt.type(0.7978845608028654) + dt.type(0.035677408136300125) * z2)
    th = jnp.tanh(u)
    s = dt.type(0.5) * z
    return s + s * th


def _tc_body(x_ref, g_ref, aux_ref, w1_ref, b1_ref, w2_ref, b2_ref,
             lnw_ref, lnb_ref, o_ref):
    xb = x_ref[...]
    xh = xb.astype(jnp.bfloat16)
    h = xh * w1_ref[...] + b1_ref[...]
    h = _gelu(h)
    d2 = jnp.dot(h, w2_ref[...], preferred_element_type=jnp.float32)
    d2 = d2 + b2_ref[...]
    d = d2[:, :_K]
    mu = d2[:, _K:_K + 1]
    c = d - mu
    var = jnp.mean(c * c, axis=-1, keepdims=True)
    dn = c * lax.rsqrt(var + _EPS) * lnw_ref[...] + lnb_ref[...]
    z = aux_ref[:, 0:1] * xb + aux_ref[:, 1:2]
    gate = 0.5 + 0.5 * jnp.tanh(0.5 * z)
    o_ref[...] = dn + g_ref[...].astype(jnp.float32) * gate


def _tc_fused(x2, g, aux, w1, b1, w2, b2, ln_w, ln_b, r_block, interpret=False):
    p_total = x2.shape[0]
    grid = (p_total // r_block,)
    w2m = jnp.mean(w2, axis=1, keepdims=True)
    w2 = jnp.concatenate(
        [w2, w2m, jnp.zeros((_HID, 7), w2.dtype)], axis=1
    ).astype(jnp.bfloat16)
    b2 = jnp.concatenate(
        [b2, jnp.mean(b2, keepdims=True), jnp.zeros((7,), b2.dtype)]
    ).reshape(1, _K + 8)
    w1 = w1.astype(jnp.bfloat16)
    b1 = b1.astype(jnp.bfloat16)
    return pl.pallas_call(
        _tc_body,
        grid=grid,
        in_specs=[
            pl.BlockSpec((r_block, 1), lambda i: (i, 0)),
            pl.BlockSpec((r_block, _K), lambda i: (i, 0)),
            pl.BlockSpec((r_block, 16), lambda i: (i, 0)),
            pl.BlockSpec((1, _HID), lambda i: (0, 0)),
            pl.BlockSpec((1, _HID), lambda i: (0, 0)),
            pl.BlockSpec((_HID, _K + 8), lambda i: (0, 0)),
            pl.BlockSpec((1, _K + 8), lambda i: (0, 0)),
            pl.BlockSpec((1, _K), lambda i: (0, 0)),
            pl.BlockSpec((1, _K), lambda i: (0, 0)),
        ],
        out_specs=pl.BlockSpec((r_block, _K), lambda i: (i, 0)),
        out_shape=jax.ShapeDtypeStruct((p_total, _K), jnp.float32),
        interpret=interpret,
    )(x2, g, aux, w1, b1.reshape(1, _HID), w2, b2,
      ln_w.reshape(1, _K), ln_b.reshape(1, _K))


def kernel(x, edge_type, mul_w, bias_w, w_edge_w, w1, b1, w2, b2, ln_w, ln_b):
    b, n, _ = x.shape
    p_total = b * n * n
    vocab = w_edge_w.shape[0]
    idx = edge_type.reshape(-1).astype(jnp.int32)
    idx2 = idx.reshape(_NW * (p_total // (_NW * _CH)), _CH)
    aux_table = jnp.concatenate(
        [mul_w, bias_w, jnp.zeros((vocab, 14), jnp.float32)], axis=1)
    g, aux = _sc_gather(w_edge_w, aux_table, idx2)
    x2 = x.reshape(p_total, 1)
    out = _tc_fused(x2, g, aux, w1, b1, w2, b2, ln_w, ln_b, r_block=512)
    return out.reshape(b, n, n, _K)
